# Initial kernel scaffold; baseline (speedup 1.0000x reference)
import jax
import jax.numpy as jnp
from jax.experimental import pallas as pl


def kernel(x1, edge_index1, batch1, x2, edge_index2, Wl1, bl1, Wr1, Wl2, bl2, Wr2, fc1_w, fc1_b, g1, be1, fc2_w, fc2_b, g2, be2, fc3_w, fc3_b):
    raise NotImplementedError("write your pallas kernel here")


# speedup vs baseline: 1.1073x; 1.1073x over previous
import functools
import jax
import jax.numpy as jnp
from jax import lax
from jax.experimental import pallas as pl
from jax.experimental.pallas import tpu as pltpu

N1 = 10000
N2 = 199
N2P = 256
E2 = 3184
F = 128
B = 16
K = 50
D2 = 64

_INTERPRET = False


def _tc_a_body(x1, msum1, cnt, Wl1, bl1, Wr1, Wl2, bl2, Wr2, p2_o, b2_o):
    mean = msum1[...] / jnp.maximum(cnt[...], 1.0)
    h = jnp.dot(mean, Wl1[...], preferred_element_type=jnp.float32)
    h = h + bl1[...] + jnp.dot(x1[...], Wr1[...], preferred_element_type=jnp.float32)
    h = jnp.maximum(h, 0.0)
    p2_o[...] = jnp.dot(h, Wl2[...], preferred_element_type=jnp.float32)
    b2_o[...] = (jnp.dot(h, Wr2[...], preferred_element_type=jnp.float32)
                 + bl2[...])


def _tc_a(x1, msum1, cnt, Wl1, bl1, Wr1, Wl2, bl2, Wr2):
    return pl.pallas_call(
        _tc_a_body,
        out_shape=[
            jax.ShapeDtypeStruct((N1, D2), jnp.float32),
            jax.ShapeDtypeStruct((N1, D2), jnp.float32),
        ],
        interpret=_INTERPRET,
    )(x1, msum1, cnt, Wl1.reshape(F, F), bl1.reshape(1, F), Wr1.reshape(F, F),
      Wl2.reshape(F, D2), bl2.reshape(1, D2), Wr2.reshape(F, D2))


def _tc_b_body(msum2, b2, cnt, src2, dst2, x2p,
               Wl1, bl1, Wr1, Wl2, bl2, Wr2, batch1,
               out1_o, out2_o, keys_o, counts_o):
    out1 = jnp.maximum(msum2[...] / jnp.maximum(cnt[...], 1.0) + b2[...], 0.0)
    out1_o[...] = out1

    cols = lax.broadcasted_iota(jnp.int32, (E2, N2P), 1)
    ohs = (cols == src2[...]).astype(jnp.float32)
    ohd = (cols == dst2[...]).astype(jnp.float32)
    A2 = jnp.dot(ohd.T, ohs, preferred_element_type=jnp.float32)
    cnt2 = jnp.maximum(jnp.sum(A2, axis=1, keepdims=True), 1.0)

    x2 = x2p[...]
    mean1 = jnp.dot(A2, x2, preferred_element_type=jnp.float32) / cnt2
    h2 = jnp.dot(mean1, Wl1[...], preferred_element_type=jnp.float32)
    h2 = h2 + bl1[...] + jnp.dot(x2, Wr1[...], preferred_element_type=jnp.float32)
    h2 = jnp.maximum(h2, 0.0)
    mean2 = jnp.dot(A2, h2, preferred_element_type=jnp.float32) / cnt2
    o2 = jnp.dot(mean2, Wl2[...], preferred_element_type=jnp.float32)
    o2 = o2 + bl2[...] + jnp.dot(h2, Wr2[...], preferred_element_type=jnp.float32)
    o2 = jnp.maximum(o2, 0.0)
    out2_o[...] = o2

    q = o2[N2 - 1:N2, :]
    sq1 = jnp.sum(out1 * out1, axis=1, keepdims=True)
    sqq = jnp.sum(q * q)
    d2 = sq1 + sqq - 2.0 * jnp.dot(out1, q.T, preferred_element_type=jnp.float32)
    keys_o[...] = jnp.sqrt(jnp.maximum(d2, 0.0) + 1e-12)

    bcols = lax.broadcasted_iota(jnp.int32, (N1, B), 1)
    counts_o[...] = jnp.sum((bcols == batch1[...]).astype(jnp.float32),
                            axis=0, keepdims=True)


def _tc_b(msum2, b2, cnt, src2, dst2, x2p, Wl1, bl1, Wr1, Wl2, bl2, Wr2, batch1):
    return pl.pallas_call(
        _tc_b_body,
        out_shape=[
            jax.ShapeDtypeStruct((N1, D2), jnp.float32),
            jax.ShapeDtypeStruct((N2P, D2), jnp.float32),
            jax.ShapeDtypeStruct((N1, 1), jnp.float32),
            jax.ShapeDtypeStruct((1, B), jnp.float32),
        ],
        interpret=_INTERPRET,
    )(msum2, b2, cnt, src2, dst2, x2p,
      Wl1.reshape(F, F), bl1.reshape(1, F), Wr1.reshape(F, F),
      Wl2.reshape(F, D2), bl2.reshape(1, D2), Wr2.reshape(F, D2), batch1)


def _tc_c_body(sel, out2p, counts_col, W3p, fc1_b, g1, be1,
               fc2_w, fc2_b, g2, be2, fc3_wp, out_o):
    o2 = out2p[...]
    sq2 = jnp.sum(o2 * o2, axis=1)[None, :]
    o2t = o2.T
    cc = counts_col[...]

    acc = jnp.zeros((B, F), jnp.float32)
    for k in range(K):
        blk = sel[k * B:(k + 1) * B, :]
        sqs = jnp.sum(blk * blk, axis=1, keepdims=True)
        d2 = sqs + sq2 - 2.0 * jnp.dot(blk, o2t, preferred_element_type=jnp.float32)
        dist = jnp.sqrt(jnp.maximum(d2, 0.0) + 1e-12)
        dist = jnp.where(cc > k, dist, 0.0)
        acc = acc + jnp.dot(dist, W3p[k * N2P:(k + 1) * N2P, :],
                            preferred_element_type=jnp.float32)

    def _ln(v, g, be):
        mu = jnp.mean(v, axis=-1, keepdims=True)
        var = jnp.mean((v - mu) ** 2, axis=-1, keepdims=True)
        return (v - mu) / jnp.sqrt(var + 1e-5) * g + be

    h = jnp.maximum(_ln(acc + fc1_b[...], g1[...], be1[...]), 0.0)
    h = jnp.dot(h, fc2_w[...], preferred_element_type=jnp.float32) + fc2_b[...]
    h = jnp.maximum(_ln(h, g2[...], be2[...]), 0.0)
    res = jnp.dot(h, fc3_wp[...], preferred_element_type=jnp.float32)
    out_o[...] = jax.nn.sigmoid(res)


def _tc_c(sel, out2p, counts_col, W3p, fc1_b, g1, be1, fc2_w, fc2_b, g2, be2,
          fc3_wp):
    return pl.pallas_call(
        _tc_c_body,
        out_shape=jax.ShapeDtypeStruct((B, F), jnp.float32),
        interpret=_INTERPRET,
    )(sel, out2p, counts_col, W3p, fc1_b.reshape(1, F), g1.reshape(1, F),
      be1.reshape(1, F), fc2_w, fc2_b.reshape(1, D2), g2.reshape(1, D2),
      be2.reshape(1, D2), fc3_wp)


def kernel(x1, edge_index1, batch1, x2, edge_index2, Wl1, bl1, Wr1, Wl2, bl2,
           Wr2, fc1_w, fc1_b, g1, be1, fc2_w, fc2_b, g2, be2, fc3_w, fc3_b):
    src1, dst1 = edge_index1[0], edge_index1[1]

    cnt = jax.ops.segment_sum(jnp.ones((src1.shape[0],), jnp.float32), dst1,
                              num_segments=N1)[:, None]
    msum1 = jax.ops.segment_sum(x1[src1], dst1, num_segments=N1)

    p2, b2 = _tc_a(x1, msum1, cnt, Wl1, bl1, Wr1, Wl2, bl2, Wr2)

    msum2 = jax.ops.segment_sum(p2[src1], dst1, num_segments=N1)

    x2p = jnp.zeros((N2P, F), jnp.float32).at[:N2].set(x2)
    src2 = edge_index2[0].reshape(E2, 1)
    dst2 = edge_index2[1].reshape(E2, 1)
    out1, out2p, keys, counts = _tc_b(
        msum2, b2, cnt, src2, dst2, x2p, Wl1, bl1, Wr1, Wl2, bl2, Wr2,
        batch1.reshape(N1, 1))

    keys1 = keys[:, 0]
    binmask = batch1[None, :] == jnp.arange(B, dtype=batch1.dtype)[:, None]
    masked = jnp.where(binmask, keys1[None, :], -jnp.inf)
    _, topi = jax.lax.top_k(masked, K)
    idx_sm = topi.T.reshape(B * K)
    sel = out1[idx_sm]

    W3p = jnp.zeros((K, N2P, F), jnp.float32).at[:, :N2, :].set(
        fc1_w.reshape(K, N2, F)).reshape(K * N2P, F)
    fc3_wp = jnp.zeros((D2, F), jnp.float32).at[:, :1].set(fc3_w)
    res = _tc_c(sel, out2p, counts.reshape(B, 1), W3p, fc1_b, g1, be1,
                fc2_w, fc2_b, g2, be2, fc3_wp)
    return res[:, :1]
